# baseline (speedup 1.0000x reference)
import functools

import jax
import jax.numpy as jnp
from jax import lax
from jax.experimental import pallas as pl
from jax.experimental.pallas import tpu as pltpu
from jax.experimental.pallas import tpu_sc as plsc

NUM_CLS = 100
HIST = NUM_CLS * NUM_CLS


def _argmax_body(ypt_ref, yt_ref, out_ref):
    x = ypt_ref[...]
    t = yt_ref[...]
    m = jnp.max(x, axis=0, keepdims=True)
    iota = lax.broadcasted_iota(jnp.int32, x.shape, 0).astype(jnp.float32)
    sel = jnp.where(x == m, iota, float(NUM_CLS))
    pred = jnp.min(sel, axis=0).astype(jnp.int32)
    out_ref[...] = t * 128 + pred


def _hist_body(nc, perw, keys_hbm, out_hbm, kv, hist):
    wid = lax.axis_index("s") * nc + lax.axis_index("c")
    base = wid * perw

    zeros16 = jnp.zeros((16,), jnp.float32)

    def zero_body(i, carry):
        for j in range(128 // 16):
            hist[i, pl.ds(j * 16, 16)] = zeros16
        return carry

    lax.fori_loop(0, NUM_CLS, zero_body, 0)

    pltpu.sync_copy(keys_hbm.at[pl.ds(base, perw)], kv)

    ones16 = jnp.ones((16,), jnp.float32)
    unroll = 4

    def body(i, carry):
        for j in range(unroll):
            k = kv[pl.ds((i * unroll + j) * 16, 16)]
            hi = k >> 7
            lo = k & 127
            plsc.addupdate_scatter(hist, [hi, lo], ones16)
        return carry

    lax.fori_loop(0, perw // (16 * unroll), body, 0)

    pltpu.sync_copy(hist, out_hbm.at[wid])


def _f1_body(*refs):
    *h_refs, o_ref = refs
    cm = sum(jnp.sum(h[...], axis=0) for h in h_refs)
    ii = lax.broadcasted_iota(jnp.int32, cm.shape, 0)
    jj = lax.broadcasted_iota(jnp.int32, cm.shape, 1)
    diag = jnp.sum(jnp.where(ii == jj, cm, 0.0), axis=1, keepdims=True)
    rows = jnp.sum(cm, axis=1, keepdims=True)
    ones_col = jnp.ones((NUM_CLS, 1), jnp.float32)
    cols = lax.dot_general(cm, ones_col, (((0,), (0,)), ((), ())),
                           preferred_element_type=jnp.float32)
    cols = lax.slice(cols, (0, 0), (NUM_CLS, 1))
    precision = diag / (rows + 1e-12)
    recall = diag / (cols + 1e-12)
    f1 = 2.0 * precision * recall / (precision + recall + 1e-12)
    o_ref[...] = jnp.sum(f1, axis=(0, 1), keepdims=True) / NUM_CLS


def kernel(y_pred, y_true):
    n, c = y_pred.shape
    assert c == NUM_CLS

    nchunk = 2
    rows = n // nchunk

    blk = 16384
    grid = rows // blk
    ypt = y_pred.T

    mesh = plsc.VectorSubcoreMesh(core_axis_name="c", subcore_axis_name="s")
    nw = mesh.num_cores * mesh.num_subcores
    perw = rows // nw
    hist_kernel = pl.kernel(
        functools.partial(_hist_body, mesh.num_cores, perw),
        out_type=jax.ShapeDtypeStruct((nw, NUM_CLS, 128), jnp.float32),
        mesh=mesh,
        scratch_types=[
            pltpu.VMEM((perw,), jnp.int32),
            pltpu.VMEM((NUM_CLS, 128), jnp.float32),
        ],
        compiler_params=pltpu.CompilerParams(needs_layout_passes=False),
    )

    hists = []
    for ci in range(nchunk):
        off = ci * grid
        keys = pl.pallas_call(
            _argmax_body,
            grid=(grid,),
            in_specs=[
                pl.BlockSpec((c, blk), lambda i, off=off: (0, off + i)),
                pl.BlockSpec((blk,), lambda i, off=off: (off + i,)),
            ],
            out_specs=pl.BlockSpec((blk,), lambda i: (i,)),
            out_shape=jax.ShapeDtypeStruct((rows,), jnp.int32),
        )(ypt, y_true)
        hists.append(hist_kernel(keys))

    out = pl.pallas_call(
        _f1_body,
        out_shape=jax.ShapeDtypeStruct((1, 1), jnp.float32),
    )(*hists)
    return out[0, 0]


# speedup vs baseline: 1.0473x; 1.0473x over previous
import functools

import jax
import jax.numpy as jnp
from jax import lax
from jax.experimental import pallas as pl
from jax.experimental.pallas import tpu as pltpu
from jax.experimental.pallas import tpu_sc as plsc

NUM_CLS = 100
HIST = NUM_CLS * NUM_CLS


def _argmax_body(ypt_ref, yt_ref, out_ref):
    x = ypt_ref[...]
    t = yt_ref[...]
    m = jnp.max(x, axis=0, keepdims=True)
    iota = lax.broadcasted_iota(jnp.int32, x.shape, 0).astype(jnp.float32)
    sel = jnp.where(x == m, iota, float(NUM_CLS))
    pred = jnp.min(sel, axis=0).astype(jnp.int32)
    out_ref[...] = t * 128 + pred


def _hist_body(nc, perw, keys_hbm, out_hbm, kv, hist):
    wid = lax.axis_index("s") * nc + lax.axis_index("c")
    base = wid * perw

    zeros16 = jnp.zeros((16,), jnp.float32)

    def zero_body(i, carry):
        for j in range(128 // 16):
            hist[i, pl.ds(j * 16, 16)] = zeros16
        return carry

    lax.fori_loop(0, NUM_CLS, zero_body, 0)

    pltpu.sync_copy(keys_hbm.at[pl.ds(base, perw)], kv)

    ones16 = jnp.ones((16,), jnp.float32)
    unroll = 4

    def body(i, carry):
        for j in range(unroll):
            k = kv[pl.ds((i * unroll + j) * 16, 16)]
            hi = k >> 7
            lo = k & 127
            plsc.addupdate_scatter(hist, [hi, lo], ones16)
        return carry

    lax.fori_loop(0, perw // (16 * unroll), body, 0)

    pltpu.sync_copy(hist, out_hbm.at[wid])


def _f1_body(*refs):
    *h_refs, o_ref = refs
    cm = sum(jnp.sum(h[...], axis=0) for h in h_refs)
    ii = lax.broadcasted_iota(jnp.int32, cm.shape, 0)
    jj = lax.broadcasted_iota(jnp.int32, cm.shape, 1)
    diag = jnp.sum(jnp.where(ii == jj, cm, 0.0), axis=1, keepdims=True)
    rows = jnp.sum(cm, axis=1, keepdims=True)
    ones_col = jnp.ones((NUM_CLS, 1), jnp.float32)
    cols = lax.dot_general(cm, ones_col, (((0,), (0,)), ((), ())),
                           preferred_element_type=jnp.float32)
    cols = lax.slice(cols, (0, 0), (NUM_CLS, 1))
    precision = diag / (rows + 1e-12)
    recall = diag / (cols + 1e-12)
    f1 = 2.0 * precision * recall / (precision + recall + 1e-12)
    o_ref[...] = jnp.sum(f1, axis=(0, 1), keepdims=True) / NUM_CLS


def kernel(y_pred, y_true):
    n, c = y_pred.shape
    assert c == NUM_CLS

    nchunk = 2
    rows = n // nchunk

    blk = 32768
    grid = rows // blk
    ypt = y_pred.T

    mesh = plsc.VectorSubcoreMesh(core_axis_name="c", subcore_axis_name="s")
    nw = mesh.num_cores * mesh.num_subcores
    perw = rows // nw
    hist_kernel = pl.kernel(
        functools.partial(_hist_body, mesh.num_cores, perw),
        out_type=jax.ShapeDtypeStruct((nw, NUM_CLS, 128), jnp.float32),
        mesh=mesh,
        scratch_types=[
            pltpu.VMEM((perw,), jnp.int32),
            pltpu.VMEM((NUM_CLS, 128), jnp.float32),
        ],
        compiler_params=pltpu.CompilerParams(needs_layout_passes=False),
    )

    hists = []
    for ci in range(nchunk):
        off = ci * grid
        keys = pl.pallas_call(
            _argmax_body,
            grid=(grid,),
            in_specs=[
                pl.BlockSpec((c, blk), lambda i, off=off: (0, off + i)),
                pl.BlockSpec((blk,), lambda i, off=off: (off + i,)),
            ],
            out_specs=pl.BlockSpec((blk,), lambda i: (i,)),
            out_shape=jax.ShapeDtypeStruct((rows,), jnp.int32),
        )(ypt, y_true)
        hists.append(hist_kernel(keys))

    out = pl.pallas_call(
        _f1_body,
        out_shape=jax.ShapeDtypeStruct((1, 1), jnp.float32),
    )(*hists)
    return out[0, 0]
